# baseline (speedup 1.0000x reference)
import functools

import jax
import jax.numpy as jnp
from jax import lax
from jax.experimental import pallas as pl
from jax.experimental.pallas import tpu as pltpu
from jax.experimental.pallas import tpu_sc as plsc

LANES = 16


def _fuse_body(token_ref, pos_ref, seg2_ref, out_ref):
    tp = token_ref[...] + pos_ref[...]
    out_ref[...] = tp[None, :, :] + seg2_ref[...][:, None, :]


def _build_fused(token_table, pos_table, seg_table):
    V, D = token_table.shape
    S = seg_table.shape[0]
    W = 2 * D
    tok2 = token_table.reshape(V // 2, W)
    pos2 = pos_table.reshape(V // 2, W)
    seg2 = jnp.concatenate([seg_table, seg_table], axis=-1)
    BV = 2000
    assert (V // 2) % BV == 0
    fused = pl.pallas_call(
        _fuse_body,
        grid=(V // 2 // BV,),
        in_specs=[
            pl.BlockSpec((BV, W), lambda i: (i, 0)),
            pl.BlockSpec((BV, W), lambda i: (i, 0)),
            pl.BlockSpec((S, W), lambda i: (0, 0)),
        ],
        out_specs=pl.BlockSpec((S, BV, W), lambda i: (0, i, 0)),
        out_shape=jax.ShapeDtypeStruct((S, V // 2, W), jnp.float32),
    )(tok2, pos2, seg2)
    return fused.reshape(S * V, D)


@functools.partial(jax.jit, static_argnums=(3,))
def _sc_gather(seq_f, lab_f, fused, V):
    N = seq_f.shape[0]
    D = fused.shape[1]
    info = plsc.get_sparse_core_info()
    NC, NS = info.num_cores, info.num_subcores
    NW = NC * NS
    assert N % NW == 0
    per_w = N // NW
    BLK = 800
    SUB = 128
    assert per_w % BLK == 0 and BLK % LANES == 0
    nblk = per_w // BLK
    subs = []
    lo = 0
    while lo < BLK:
        subs.append((lo, min(SUB, BLK - lo)))
        lo += SUB
    mesh = plsc.VectorSubcoreMesh(core_axis_name="c", subcore_axis_name="s")

    assert nblk % 2 == 0

    @functools.partial(
        pl.kernel,
        mesh=mesh,
        compiler_params=pltpu.CompilerParams(use_tc_tiling_on_sc=False),
        out_type=jax.ShapeDtypeStruct((N, 2 * D), jnp.float32),
        scratch_types=[
            pltpu.VMEM((BLK,), jnp.int32),
            pltpu.VMEM((BLK,), jnp.int32),
            pltpu.VMEM((BLK,), jnp.int32),
            pltpu.VMEM((BLK,), jnp.int32),
            pltpu.VMEM((BLK,), jnp.int32),
            pltpu.VMEM((BLK,), jnp.int32),
            pltpu.VMEM((BLK, D), jnp.float32),
            pltpu.VMEM((BLK, D), jnp.float32),
            pltpu.SemaphoreType.DMA,
            pltpu.SemaphoreType.DMA,
            pltpu.SemaphoreType.DMA,
            pltpu.SemaphoreType.DMA,
        ],
    )
    def gather(seq_hbm, lab_hbm, table_hbm, out_hbm, seq_a, seq_b, lab_a,
               lab_b, idx_a, idx_b, rows_a, rows_b, sio_a, sio_b, sg_a,
               sg_b):
        wid = lax.axis_index("s") * NC + lax.axis_index("c")
        w0 = wid * per_w
        S = (seq_a, seq_b)
        A = (lab_a, lab_b)
        I = (idx_a, idx_b)
        R = (rows_a, rows_b)
        SIO = (sio_a, sio_b)
        SG = (sg_a, sg_b)

        def fire_io(b, p):
            base = w0 + b * BLK
            pltpu.async_copy(seq_hbm.at[pl.ds(base, BLK)], S[p], SIO[p])
            pltpu.async_copy(lab_hbm.at[pl.ds(base, BLK)], A[p], SIO[p])

        def wait_io(b, p):
            base = w0 + b * BLK
            pltpu.make_async_copy(seq_hbm.at[pl.ds(base, BLK)], S[p],
                                  SIO[p]).wait()
            pltpu.make_async_copy(lab_hbm.at[pl.ds(base, BLK)], A[p],
                                  SIO[p]).wait()

        def calc_idx(p):
            def cidx(i, c):
                sl = pl.ds(i * LANES, LANES)
                I[p][sl] = A[p][sl] * V + S[p][sl]
                return c

            lax.fori_loop(0, BLK // LANES, cidx, 0)

        def fire_gathers(p):
            for lo, n in subs:
                sl = pl.ds(lo, n)
                pltpu.async_copy(table_hbm.at[I[p].at[sl]], R[p].at[sl],
                                 SG[p])

        def wait_gathers(p):
            for lo, n in subs:
                sl = pl.ds(lo, n)
                pltpu.make_async_copy(table_hbm.at[I[p].at[sl]],
                                      R[p].at[sl], SG[p]).wait()

        def out_copy(b, p):
            base = w0 + b * BLK
            pltpu.sync_copy(R[p], out_hbm.at[pl.ds(base, BLK), pl.ds(0, D)])

        fire_io(0, 0)
        wait_io(0, 0)
        calc_idx(0)
        fire_gathers(0)
        fire_io(1, 1)

        def body(g, carry):
            b1 = 2 * g + 1
            wait_io(b1, 1)
            calc_idx(1)
            fire_gathers(1)
            fire_io(b1 + 1, 0)
            wait_gathers(0)
            out_copy(b1 - 1, 0)
            b2 = b1 + 1
            wait_io(b2, 0)
            calc_idx(0)
            fire_gathers(0)

            @pl.when(b2 + 1 < nblk)
            def _():
                fire_io(b2 + 1, 1)

            wait_gathers(1)
            out_copy(b2 - 1, 1)
            return carry

        lax.fori_loop(0, nblk // 2 - 1, body, 0)
        bl = nblk - 1
        wait_io(bl, 1)
        calc_idx(1)
        fire_gathers(1)
        wait_gathers(0)
        out_copy(bl - 1, 0)
        wait_gathers(1)
        out_copy(bl, 1)

    return gather(seq_f, lab_f, fused)


def kernel(seq, segment_lab, token_table, pos_table, seg_table):
    B, L = seq.shape
    V, D = token_table.shape
    fused = _build_fused(token_table, pos_table, seg_table)
    out = _sc_gather(seq.reshape(-1), segment_lab.reshape(-1), fused, V)
    return out[:, :D].reshape(B, L, D)


# speedup vs baseline: 1.0035x; 1.0035x over previous
import functools

import jax
import jax.numpy as jnp
from jax import lax
from jax.experimental import pallas as pl
from jax.experimental.pallas import tpu as pltpu
from jax.experimental.pallas import tpu_sc as plsc

LANES = 16


def _fuse_body(token_ref, pos_ref, seg2_ref, out_ref):
    tp = token_ref[...] + pos_ref[...]
    out_ref[...] = tp[None, :, :] + seg2_ref[...][:, None, :]


def _build_fused(token_table, pos_table, seg_table):
    V, D = token_table.shape
    S = seg_table.shape[0]
    W = 2 * D
    tok2 = token_table.reshape(V // 2, W)
    pos2 = pos_table.reshape(V // 2, W)
    seg2 = jnp.concatenate([seg_table, seg_table], axis=-1)
    BV = 2000
    assert (V // 2) % BV == 0
    fused = pl.pallas_call(
        _fuse_body,
        grid=(V // 2 // BV,),
        in_specs=[
            pl.BlockSpec((BV, W), lambda i: (i, 0)),
            pl.BlockSpec((BV, W), lambda i: (i, 0)),
            pl.BlockSpec((S, W), lambda i: (0, 0)),
        ],
        out_specs=pl.BlockSpec((S, BV, W), lambda i: (0, i, 0)),
        out_shape=jax.ShapeDtypeStruct((S, V // 2, W), jnp.float32),
    )(tok2, pos2, seg2)
    return fused.reshape(S * V, D)


@functools.partial(jax.jit, static_argnums=(3,))
def _sc_gather(seq_f, lab_f, fused, V):
    N = seq_f.shape[0]
    D = fused.shape[1]
    info = plsc.get_sparse_core_info()
    NC, NS = info.num_cores, info.num_subcores
    NW = NC * NS
    assert N % NW == 0
    per_w = N // NW
    BLK = 640
    SUB = 128
    assert per_w % BLK == 0 and BLK % LANES == 0
    nblk = per_w // BLK
    subs = []
    lo = 0
    while lo < BLK:
        subs.append((lo, min(SUB, BLK - lo)))
        lo += SUB
    mesh = plsc.VectorSubcoreMesh(core_axis_name="c", subcore_axis_name="s")

    assert nblk % 2 == 0

    @functools.partial(
        pl.kernel,
        mesh=mesh,
        compiler_params=pltpu.CompilerParams(use_tc_tiling_on_sc=False),
        out_type=jax.ShapeDtypeStruct((N, 2 * D), jnp.float32),
        scratch_types=[
            pltpu.VMEM((BLK,), jnp.int32),
            pltpu.VMEM((BLK,), jnp.int32),
            pltpu.VMEM((BLK,), jnp.int32),
            pltpu.VMEM((BLK,), jnp.int32),
            pltpu.VMEM((BLK,), jnp.int32),
            pltpu.VMEM((BLK,), jnp.int32),
            pltpu.VMEM((BLK, D), jnp.float32),
            pltpu.VMEM((BLK, D), jnp.float32),
            pltpu.SemaphoreType.DMA,
            pltpu.SemaphoreType.DMA,
            pltpu.SemaphoreType.DMA,
            pltpu.SemaphoreType.DMA,
        ],
    )
    def gather(seq_hbm, lab_hbm, table_hbm, out_hbm, seq_a, seq_b, lab_a,
               lab_b, idx_a, idx_b, rows_a, rows_b, sio_a, sio_b, sg_a,
               sg_b):
        wid = lax.axis_index("s") * NC + lax.axis_index("c")
        w0 = wid * per_w
        S = (seq_a, seq_b)
        A = (lab_a, lab_b)
        I = (idx_a, idx_b)
        R = (rows_a, rows_b)
        SIO = (sio_a, sio_b)
        SG = (sg_a, sg_b)

        def fire_io(b, p):
            base = w0 + b * BLK
            pltpu.async_copy(seq_hbm.at[pl.ds(base, BLK)], S[p], SIO[p])
            pltpu.async_copy(lab_hbm.at[pl.ds(base, BLK)], A[p], SIO[p])

        def wait_io(b, p):
            base = w0 + b * BLK
            pltpu.make_async_copy(seq_hbm.at[pl.ds(base, BLK)], S[p],
                                  SIO[p]).wait()
            pltpu.make_async_copy(lab_hbm.at[pl.ds(base, BLK)], A[p],
                                  SIO[p]).wait()

        def calc_idx(p):
            def cidx(i, c):
                sl = pl.ds(i * LANES, LANES)
                I[p][sl] = A[p][sl] * V + S[p][sl]
                return c

            lax.fori_loop(0, BLK // LANES, cidx, 0)

        def fire_gathers(p):
            for lo, n in subs:
                sl = pl.ds(lo, n)
                pltpu.async_copy(table_hbm.at[I[p].at[sl]], R[p].at[sl],
                                 SG[p])

        def wait_gathers(p):
            for lo, n in subs:
                sl = pl.ds(lo, n)
                pltpu.make_async_copy(table_hbm.at[I[p].at[sl]],
                                      R[p].at[sl], SG[p]).wait()

        def out_copy(b, p):
            base = w0 + b * BLK
            pltpu.sync_copy(R[p], out_hbm.at[pl.ds(base, BLK), pl.ds(0, D)])

        fire_io(0, 0)
        wait_io(0, 0)
        calc_idx(0)
        fire_gathers(0)
        fire_io(1, 1)

        def body(g, carry):
            b1 = 2 * g + 1
            wait_io(b1, 1)
            calc_idx(1)
            fire_gathers(1)
            fire_io(b1 + 1, 0)
            wait_gathers(0)
            out_copy(b1 - 1, 0)
            b2 = b1 + 1
            wait_io(b2, 0)
            calc_idx(0)
            fire_gathers(0)

            @pl.when(b2 + 1 < nblk)
            def _():
                fire_io(b2 + 1, 1)

            wait_gathers(1)
            out_copy(b2 - 1, 1)
            return carry

        lax.fori_loop(0, nblk // 2 - 1, body, 0)
        bl = nblk - 1
        wait_io(bl, 1)
        calc_idx(1)
        fire_gathers(1)
        wait_gathers(0)
        out_copy(bl - 1, 0)
        wait_gathers(1)
        out_copy(bl, 1)

    return gather(seq_f, lab_f, fused)


def kernel(seq, segment_lab, token_table, pos_table, seg_table):
    B, L = seq.shape
    V, D = token_table.shape
    fused = _build_fused(token_table, pos_table, seg_table)
    out = _sc_gather(seq.reshape(-1), segment_lab.reshape(-1), fused, V)
    return out[:, :D].reshape(B, L, D)
